# baseline (speedup 1.0000x reference)
import functools

import jax
import jax.numpy as jnp
from jax import lax
from jax.experimental import pallas as pl
from jax.experimental.pallas import tpu as pltpu
from jax.experimental.pallas import tpu_sc as plsc

N = 10000
E = 320000
F_IN = 128
HID = 64
ED = 15
G = 64

NP = 10240
NW = 32
EPW = E // NW
K = 80
NCHUNK = EPW // K
RPT = NP // 16


CH = 5
NSUP = NCHUNK // CH


def _sc_edge_body(ei_hbm, ae_hbm, asrc_hbm, adst_hbm, h_hbm,
                  acc_out, den_out,
                  asrc_v, adst_v, eidx, ae5, exA, exB, exC,
                  rowsA, rowsB, rowsC, den_v, red_v, tmp_v, acc_sh, den_sh,
                  sg0, sg1, sg2, ss0, ss1, ss2):
    rows = [rowsA, rowsB, rowsC]
    exb = [exA, exB, exC]
    semg = [sg0, sg1, sg2]
    sems = [ss0, ss1, ss2]
    c = lax.axis_index("c")
    s = lax.axis_index("s")
    wid = s * 2 + c

    pltpu.sync_copy(asrc_hbm, asrc_v.at[pl.ds(0, N)])
    pltpu.sync_copy(adst_hbm, adst_v.at[pl.ds(0, N)])

    zero16 = jnp.zeros((16,), jnp.float32)

    @plsc.parallel_loop(0, NP // 16)
    def _zden(i):
        den_v[pl.ds(i * 16, 16)] = zero16

    @plsc.parallel_loop(0, K)
    def _zrows(i):
        for j in range(HID // 16):
            rowsA[i, pl.ds(j * 16, 16)] = zero16

    for j in range(RPT // K):
        pltpu.sync_copy(rowsA, acc_sh.at[pl.ds(s * RPT + j * K, K), :])
    plsc.subcore_barrier()

    def _sup(si, _):
        sbase = wid * NCHUNK + si * CH
        pltpu.sync_copy(ei_hbm.at[:, pl.ds(sbase, CH), :], eidx)
        pltpu.sync_copy(ae_hbm.at[pl.ds(sbase * K, CH * K)], ae5)
        gd = [None, None, None]
        sd = [None, None, None]
        gd[0] = pltpu.async_copy(h_hbm.at[eidx.at[0, 0]], rows[0], semg[0])
        for j in range(CH):
            b = j % 3
            nb = (j + 1) % 3
            if j + 1 < CH:
                if sd[nb] is not None:
                    sd[nb].wait()
                gd[nb] = pltpu.async_copy(h_hbm.at[eidx.at[0, j + 1]],
                                          rows[nb], semg[nb])
            exv = exb[b]
            for g in range(K // 16):
                sl = pl.ds(g * 16, 16)
                srcg = eidx[0, j, sl]
                dstg = eidx[1, j, sl]
                a = (plsc.load_gather(asrc_v, [srcg])
                     + plsc.load_gather(adst_v, [dstg])
                     + ae5[pl.ds(j * K + g * 16, 16)])
                a = jnp.where(a > 0, a, 0.2 * a)
                exg = jnp.exp(a)
                plsc.addupdate_scatter(den_v, [dstg], exg)
                exv[sl] = exg
            gd[b].wait()
            rv = rows[b]

            @plsc.parallel_loop(0, K // 16)
            def _scale(gi):
                exg = exv[pl.ds(gi * 16, 16)]
                for i in range(16):
                    e = exg[i]
                    r = gi * 16 + i
                    for jj in range(HID // 16):
                        sl2 = pl.ds(jj * 16, 16)
                        rv[r, sl2] = rv[r, sl2] * e
            sd[b] = pltpu.async_copy(rv, acc_sh.at[eidx.at[1, j]],
                                     sems[b], add=True)
        for b in range(3):
            if sd[b] is not None:
                sd[b].wait()
        return ()
    lax.fori_loop(0, NSUP, _sup, ())

    plsc.subcore_barrier()

    pltpu.sync_copy(den_v, den_sh.at[s])
    plsc.subcore_barrier()

    rbase = s * RPT
    pltpu.sync_copy(den_sh.at[0, pl.ds(rbase, RPT)], red_v)
    for k2 in range(1, 16):
        pltpu.sync_copy(den_sh.at[k2, pl.ds(rbase, RPT)], tmp_v)

        @plsc.parallel_loop(0, RPT // 16)
        def _acc(g):
            sl = pl.ds(g * 16, 16)
            red_v[sl] = red_v[sl] + tmp_v[sl]

    pltpu.sync_copy(red_v, den_out.at[c, pl.ds(rbase, RPT)])
    pltpu.sync_copy(acc_sh.at[pl.ds(rbase, RPT), :],
                    acc_out.at[c, pl.ds(rbase, RPT), :])


_sc_edge = functools.partial(
    pl.kernel,
    out_type=[jax.ShapeDtypeStruct((2, NP, HID), jnp.float32),
              jax.ShapeDtypeStruct((2, NP), jnp.float32)],
    mesh=plsc.VectorSubcoreMesh(core_axis_name="c", subcore_axis_name="s"),
    compiler_params=pltpu.CompilerParams(needs_layout_passes=False,
                                         use_tc_tiling_on_sc=False),
    scratch_types=[
        pltpu.VMEM((NP,), jnp.float32),
        pltpu.VMEM((NP,), jnp.float32),
        pltpu.VMEM((2, CH, K), jnp.int32),
        pltpu.VMEM((CH * K,), jnp.float32),
        pltpu.VMEM((K,), jnp.float32),
        pltpu.VMEM((K,), jnp.float32),
        pltpu.VMEM((K,), jnp.float32),
        pltpu.VMEM((K, HID), jnp.float32),
        pltpu.VMEM((K, HID), jnp.float32),
        pltpu.VMEM((K, HID), jnp.float32),
        pltpu.VMEM((NP,), jnp.float32),
        pltpu.VMEM((RPT,), jnp.float32),
        pltpu.VMEM((RPT,), jnp.float32),
        pltpu.VMEM_SHARED((NP, HID), jnp.float32),
        pltpu.VMEM_SHARED((16, NP), jnp.float32),
        pltpu.SemaphoreType.DMA,
        pltpu.SemaphoreType.DMA,
        pltpu.SemaphoreType.DMA,
        pltpu.SemaphoreType.DMA,
        pltpu.SemaphoreType.DMA,
        pltpu.SemaphoreType.DMA,
    ],
)(_sc_edge_body)


_BLK = 2000


_MBLK = 16000


def _tc_pre_body(x_ref, w_ref, aw_ref, dw_ref, w1t_ref, a1_ref,
                 w2t_ref, a2_ref, ea_ref,
                 h_ref, as_ref, ad_ref, v_ref, o1_ref, o2_ref):
    i = pl.program_id(0)

    @pl.when(i == 0)
    def _():
        v1 = jnp.dot(a1_ref[...], w1t_ref[...],
                     preferred_element_type=jnp.float32)
        v2 = jnp.dot(a2_ref[...], w2t_ref[...],
                     preferred_element_type=jnp.float32)
        v_ref[...] = jnp.concatenate([v1, v2], axis=0)

    @pl.when(i < N // _BLK)
    def _():
        h = jnp.dot(x_ref[...], w_ref[...],
                    preferred_element_type=jnp.float32)
        h_ref[...] = h
        as_ref[...] = jnp.dot(h, aw_ref[...],
                              preferred_element_type=jnp.float32)
        ad_ref[...] = jnp.dot(h, dw_ref[...],
                              preferred_element_type=jnp.float32)

    ea = ea_ref[...]
    v = v_ref[...]
    o1_ref[...] = jnp.sum(ea * v[0:1, :], axis=1, keepdims=True)
    o2_ref[...] = jnp.sum(ea * v[1:2, :], axis=1, keepdims=True)


def _tc_pre(x_in, w, att_s, att_d, we1t, ae1, we2t, ae2, ea):
    nnode = N // _BLK

    def capped(i):
        return (jnp.minimum(i, nnode - 1), 0)

    return pl.pallas_call(
        _tc_pre_body,
        grid=(E // _MBLK,),
        in_specs=[
            pl.BlockSpec((_BLK, F_IN), capped),
            pl.BlockSpec((F_IN, HID), lambda i: (0, 0)),
            pl.BlockSpec((HID, 1), lambda i: (0, 0)),
            pl.BlockSpec((HID, 1), lambda i: (0, 0)),
            pl.BlockSpec((HID, ED), lambda i: (0, 0)),
            pl.BlockSpec((1, HID), lambda i: (0, 0)),
            pl.BlockSpec((HID, ED), lambda i: (0, 0)),
            pl.BlockSpec((1, HID), lambda i: (0, 0)),
            pl.BlockSpec((_MBLK, ED), lambda i: (i, 0)),
        ],
        out_specs=[
            pl.BlockSpec((_BLK, HID), capped),
            pl.BlockSpec((_BLK, 1), capped),
            pl.BlockSpec((_BLK, 1), capped),
            pl.BlockSpec((2, ED), lambda i: (0, 0)),
            pl.BlockSpec((_MBLK, 1), lambda i: (i, 0)),
            pl.BlockSpec((_MBLK, 1), lambda i: (i, 0)),
        ],
        out_shape=[
            jax.ShapeDtypeStruct((N, HID), jnp.float32),
            jax.ShapeDtypeStruct((N, 1), jnp.float32),
            jax.ShapeDtypeStruct((N, 1), jnp.float32),
            jax.ShapeDtypeStruct((2, ED), jnp.float32),
            jax.ShapeDtypeStruct((E, 1), jnp.float32),
            jax.ShapeDtypeStruct((E, 1), jnp.float32),
        ],
    )(x_in, w, att_s.reshape(HID, 1), att_d.reshape(HID, 1),
      we1t, ae1.reshape(1, HID), we2t, ae2.reshape(1, HID), ea)


def _tc_mid_body(a0, a1, d0, d1, b_ref, w_ref, aw_ref, dw_ref,
                 h_ref, as_ref, ad_ref):
    num = a0[0] + a1[0]
    den = d0[0] + d1[0]
    x2 = num / (den + 1e-16) + b_ref[...]
    x2 = jnp.maximum(x2, 0.0)
    h = jnp.dot(x2, w_ref[...], preferred_element_type=jnp.float32)
    h_ref[...] = h
    as_ref[...] = jnp.dot(h, aw_ref[...], preferred_element_type=jnp.float32)
    ad_ref[...] = jnp.dot(h, dw_ref[...], preferred_element_type=jnp.float32)


def _tc_mid(acc, den3, b, w, att_s, att_d):
    return pl.pallas_call(
        _tc_mid_body,
        grid=(N // _BLK,),
        in_specs=[
            pl.BlockSpec((1, _BLK, HID), lambda i: (0, i, 0)),
            pl.BlockSpec((1, _BLK, HID), lambda i: (1, i, 0)),
            pl.BlockSpec((1, _BLK, 1), lambda i: (0, i, 0)),
            pl.BlockSpec((1, _BLK, 1), lambda i: (1, i, 0)),
            pl.BlockSpec((1, HID), lambda i: (0, 0)),
            pl.BlockSpec((HID, HID), lambda i: (0, 0)),
            pl.BlockSpec((HID, 1), lambda i: (0, 0)),
            pl.BlockSpec((HID, 1), lambda i: (0, 0)),
        ],
        out_specs=[
            pl.BlockSpec((_BLK, HID), lambda i: (i, 0)),
            pl.BlockSpec((_BLK, 1), lambda i: (i, 0)),
            pl.BlockSpec((_BLK, 1), lambda i: (i, 0)),
        ],
        out_shape=[
            jax.ShapeDtypeStruct((N, HID), jnp.float32),
            jax.ShapeDtypeStruct((N, 1), jnp.float32),
            jax.ShapeDtypeStruct((N, 1), jnp.float32),
        ],
    )(acc, acc, den3, den3, b.reshape(1, HID), w,
      att_s.reshape(HID, 1), att_d.reshape(HID, 1))


def _tc_final_body(acc_ref, den_ref, b_ref, bt_ref, wl_ref, bl_ref, out_ref):
    h = ((acc_ref[0] + acc_ref[1])
         / (den_ref[0] + den_ref[1] + 1e-16) + b_ref[...])
    ids = bt_ref[...]
    gidx = lax.broadcasted_iota(jnp.int32, (G, NP), 0)
    mask = (ids == gidx).astype(jnp.float32)
    sums = jnp.dot(mask, h, preferred_element_type=jnp.float32)
    cnt = jnp.sum(mask, axis=1, keepdims=True)
    pooled = sums / jnp.maximum(cnt, 1.0)
    out_ref[...] = jnp.tanh(
        jnp.dot(pooled, wl_ref[...], preferred_element_type=jnp.float32)
        + bl_ref[...])


def _tc_final(acc, den3, b, batch_p, wl, bl):
    return pl.pallas_call(
        _tc_final_body,
        out_shape=jax.ShapeDtypeStruct((G, 1), jnp.float32),
    )(acc, den3, b.reshape(1, HID), batch_p.reshape(1, NP), wl,
      bl.reshape(1, 1))


def kernel(x, edge_index, edge_attr, batch, W1, att_src1, att_dst1, We1,
           att_e1, b1, W2, att_src2, att_dst2, We2, att_e2, b2, Wl, bl):
    batch_p = jnp.pad(batch, (0, NP - N), constant_values=G)
    ei3 = edge_index.reshape(2, E // K, K)

    h1, as1, ad1, _v12, ae1, ae2 = _tc_pre(
        x, W1, att_src1, att_dst1, We1.T, att_e1, We2.T, att_e2, edge_attr)
    acc1, den1 = _sc_edge(ei3, ae1.reshape(E), as1.reshape(N),
                          ad1.reshape(N), h1)

    h2, as2, ad2 = _tc_mid(acc1, den1.reshape(2, NP, 1), b1, W2,
                           att_src2, att_dst2)
    acc2, den2 = _sc_edge(ei3, ae2.reshape(E), as2.reshape(N),
                          ad2.reshape(N), h2)

    return _tc_final(acc2, den2.reshape(2, NP, 1), b2, batch_p, Wl, bl)


# speedup vs baseline: 1.3741x; 1.3741x over previous
import functools

import jax
import jax.numpy as jnp
from jax import lax
from jax.experimental import pallas as pl
from jax.experimental.pallas import tpu as pltpu
from jax.experimental.pallas import tpu_sc as plsc

N = 10000
E = 320000
F_IN = 128
HID = 64
ED = 15
G = 64

NP = 10240
NW = 32
EPW = E // NW
K = 80
NCHUNK = EPW // K
RPT = NP // 16


CH = 5
NSUP = NCHUNK // CH


def _sc_edge_body(ei_hbm, ae_hbm, asrc_hbm, adst_hbm, h_hbm,
                  acc_out, den_out,
                  asrc_v, adst_v, eidx, ae5, exA, exB, exC,
                  rowsA, rowsB, rowsC, den_v, red_v, tmp_v, acc_sh, den_sh,
                  sg0, sg1, sg2, ss0, ss1, ss2):
    rows = [rowsA, rowsB, rowsC]
    exb = [exA, exB, exC]
    semg = [sg0, sg1, sg2]
    sems = [ss0, ss1, ss2]
    c = lax.axis_index("c")
    s = lax.axis_index("s")
    wid = s * 2 + c

    pltpu.sync_copy(asrc_hbm, asrc_v.at[pl.ds(0, N)])
    pltpu.sync_copy(adst_hbm, adst_v.at[pl.ds(0, N)])

    zero16 = jnp.zeros((16,), jnp.float32)

    @plsc.parallel_loop(0, NP // 16)
    def _zden(i):
        den_v[pl.ds(i * 16, 16)] = zero16

    @plsc.parallel_loop(0, K)
    def _zrows(i):
        for j in range(HID // 16):
            rowsA[i, pl.ds(j * 16, 16)] = zero16

    for j in range(RPT // K):
        pltpu.sync_copy(rowsA, acc_sh.at[pl.ds(s * RPT + j * K, K), :])
    plsc.subcore_barrier()

    def _sup(si, _):
        sbase = wid * NCHUNK + si * CH
        pltpu.sync_copy(ei_hbm.at[:, pl.ds(sbase, CH), :], eidx)
        pltpu.sync_copy(ae_hbm.at[pl.ds(sbase, CH), :], ae5)
        gd = [None, None, None]
        sd = [None, None, None]
        gd[0] = pltpu.async_copy(h_hbm.at[eidx.at[0, 0]], rows[0], semg[0])
        for j in range(CH):
            b = j % 3
            nb = (j + 1) % 3
            if j + 1 < CH:
                if sd[nb] is not None:
                    sd[nb].wait()
                gd[nb] = pltpu.async_copy(h_hbm.at[eidx.at[0, j + 1]],
                                          rows[nb], semg[nb])
            exv = exb[b]
            for g in range(K // 16):
                sl = pl.ds(g * 16, 16)
                srcg = eidx[0, j, sl]
                dstg = eidx[1, j, sl]
                a = (plsc.load_gather(asrc_v, [srcg])
                     + plsc.load_gather(adst_v, [dstg])
                     + ae5[j, pl.ds(g * 16, 16)])
                a = jnp.where(a > 0, a, 0.2 * a)
                exg = jnp.exp(a)
                plsc.addupdate_scatter(den_v, [dstg], exg)
                exv[sl] = exg
            gd[b].wait()
            rv = rows[b]

            @plsc.parallel_loop(0, K // 16)
            def _scale(gi):
                exg = exv[pl.ds(gi * 16, 16)]
                for i in range(16):
                    e = exg[i]
                    r = gi * 16 + i
                    for jj in range(HID // 16):
                        sl2 = pl.ds(jj * 16, 16)
                        rv[r, sl2] = rv[r, sl2] * e
            sd[b] = pltpu.async_copy(rv, acc_sh.at[eidx.at[1, j]],
                                     sems[b], add=True)
        for b in range(3):
            if sd[b] is not None:
                sd[b].wait()
        return ()
    lax.fori_loop(0, NSUP, _sup, ())

    plsc.subcore_barrier()

    pltpu.sync_copy(den_v, den_sh.at[s])
    plsc.subcore_barrier()

    rbase = s * RPT
    pltpu.sync_copy(den_sh.at[0, pl.ds(rbase, RPT)], red_v)
    for k2 in range(1, 16):
        pltpu.sync_copy(den_sh.at[k2, pl.ds(rbase, RPT)], tmp_v)

        @plsc.parallel_loop(0, RPT // 16)
        def _acc(g):
            sl = pl.ds(g * 16, 16)
            red_v[sl] = red_v[sl] + tmp_v[sl]

    pltpu.sync_copy(red_v, den_out.at[c, pl.ds(rbase, RPT)])
    pltpu.sync_copy(acc_sh.at[pl.ds(rbase, RPT), :],
                    acc_out.at[c, pl.ds(rbase, RPT), :])


_sc_edge = functools.partial(
    pl.kernel,
    out_type=[jax.ShapeDtypeStruct((2, NP, HID), jnp.float32),
              jax.ShapeDtypeStruct((2, NP), jnp.float32)],
    mesh=plsc.VectorSubcoreMesh(core_axis_name="c", subcore_axis_name="s"),
    compiler_params=pltpu.CompilerParams(needs_layout_passes=False,
                                         use_tc_tiling_on_sc=False),
    scratch_types=[
        pltpu.VMEM((NP,), jnp.float32),
        pltpu.VMEM((NP,), jnp.float32),
        pltpu.VMEM((2, CH, K), jnp.int32),
        pltpu.VMEM((CH, K), jnp.float32),
        pltpu.VMEM((K,), jnp.float32),
        pltpu.VMEM((K,), jnp.float32),
        pltpu.VMEM((K,), jnp.float32),
        pltpu.VMEM((K, HID), jnp.float32),
        pltpu.VMEM((K, HID), jnp.float32),
        pltpu.VMEM((K, HID), jnp.float32),
        pltpu.VMEM((NP,), jnp.float32),
        pltpu.VMEM((RPT,), jnp.float32),
        pltpu.VMEM((RPT,), jnp.float32),
        pltpu.VMEM_SHARED((NP, HID), jnp.float32),
        pltpu.VMEM_SHARED((16, NP), jnp.float32),
        pltpu.SemaphoreType.DMA,
        pltpu.SemaphoreType.DMA,
        pltpu.SemaphoreType.DMA,
        pltpu.SemaphoreType.DMA,
        pltpu.SemaphoreType.DMA,
        pltpu.SemaphoreType.DMA,
    ],
)(_sc_edge_body)


_BLK = 2000


def _tc_node_body(x_ref, w_ref, aw_ref, dw_ref, w1t_ref, a1_ref,
                  w2t_ref, a2_ref, h_ref, as_ref, ad_ref, v_ref):
    h = jnp.dot(x_ref[...], w_ref[...], preferred_element_type=jnp.float32)
    h_ref[...] = h
    as_ref[...] = jnp.dot(h, aw_ref[...], preferred_element_type=jnp.float32)
    ad_ref[...] = jnp.dot(h, dw_ref[...], preferred_element_type=jnp.float32)
    v1 = jnp.dot(a1_ref[...], w1t_ref[...], preferred_element_type=jnp.float32)
    v2 = jnp.dot(a2_ref[...], w2t_ref[...], preferred_element_type=jnp.float32)
    v_ref[...] = jnp.concatenate([v1, v2], axis=0)


def _tc_node(x_in, w, att_s, att_d, we1t, ae1, we2t, ae2):
    return pl.pallas_call(
        _tc_node_body,
        grid=(N // _BLK,),
        in_specs=[
            pl.BlockSpec((_BLK, F_IN), lambda i: (i, 0)),
            pl.BlockSpec((F_IN, HID), lambda i: (0, 0)),
            pl.BlockSpec((HID, 1), lambda i: (0, 0)),
            pl.BlockSpec((HID, 1), lambda i: (0, 0)),
            pl.BlockSpec((HID, ED), lambda i: (0, 0)),
            pl.BlockSpec((1, HID), lambda i: (0, 0)),
            pl.BlockSpec((HID, ED), lambda i: (0, 0)),
            pl.BlockSpec((1, HID), lambda i: (0, 0)),
        ],
        out_specs=[
            pl.BlockSpec((_BLK, HID), lambda i: (i, 0)),
            pl.BlockSpec((_BLK, 1), lambda i: (i, 0)),
            pl.BlockSpec((_BLK, 1), lambda i: (i, 0)),
            pl.BlockSpec((2, ED), lambda i: (0, 0)),
        ],
        out_shape=[
            jax.ShapeDtypeStruct((N, HID), jnp.float32),
            jax.ShapeDtypeStruct((N, 1), jnp.float32),
            jax.ShapeDtypeStruct((N, 1), jnp.float32),
            jax.ShapeDtypeStruct((2, ED), jnp.float32),
        ],
    )(x_in, w, att_s.reshape(HID, 1), att_d.reshape(HID, 1),
      we1t, ae1.reshape(1, HID), we2t, ae2.reshape(1, HID))


_EB2 = 200


def _tc_edge_body(ea_ref, v_ref, o1_ref, o2_ref):
    ea = ea_ref[...]
    v1 = v_ref[0:1, :].reshape(1, 1, ED)
    v2 = v_ref[1:2, :].reshape(1, 1, ED)
    o1_ref[...] = jnp.sum(ea * v1, axis=2)
    o2_ref[...] = jnp.sum(ea * v2, axis=2)


def _tc_edge(ea3, v12):
    nrow = E // K
    return pl.pallas_call(
        _tc_edge_body,
        grid=(nrow // _EB2,),
        in_specs=[
            pl.BlockSpec((_EB2, K, ED), lambda i: (i, 0, 0)),
            pl.BlockSpec((2, ED), lambda i: (0, 0)),
        ],
        out_specs=[
            pl.BlockSpec((_EB2, K), lambda i: (i, 0)),
            pl.BlockSpec((_EB2, K), lambda i: (i, 0)),
        ],
        out_shape=[
            jax.ShapeDtypeStruct((nrow, K), jnp.float32),
            jax.ShapeDtypeStruct((nrow, K), jnp.float32),
        ],
    )(ea3, v12)


def _tc_mid_body(a0, a1, d0, d1, b_ref, w_ref, aw_ref, dw_ref,
                 h_ref, as_ref, ad_ref):
    num = a0[0] + a1[0]
    den = d0[0] + d1[0]
    x2 = num / (den + 1e-16) + b_ref[...]
    x2 = jnp.maximum(x2, 0.0)
    h = jnp.dot(x2, w_ref[...], preferred_element_type=jnp.float32)
    h_ref[...] = h
    as_ref[...] = jnp.dot(h, aw_ref[...], preferred_element_type=jnp.float32)
    ad_ref[...] = jnp.dot(h, dw_ref[...], preferred_element_type=jnp.float32)


def _tc_mid(acc, den3, b, w, att_s, att_d):
    return pl.pallas_call(
        _tc_mid_body,
        grid=(N // _BLK,),
        in_specs=[
            pl.BlockSpec((1, _BLK, HID), lambda i: (0, i, 0)),
            pl.BlockSpec((1, _BLK, HID), lambda i: (1, i, 0)),
            pl.BlockSpec((1, _BLK, 1), lambda i: (0, i, 0)),
            pl.BlockSpec((1, _BLK, 1), lambda i: (1, i, 0)),
            pl.BlockSpec((1, HID), lambda i: (0, 0)),
            pl.BlockSpec((HID, HID), lambda i: (0, 0)),
            pl.BlockSpec((HID, 1), lambda i: (0, 0)),
            pl.BlockSpec((HID, 1), lambda i: (0, 0)),
        ],
        out_specs=[
            pl.BlockSpec((_BLK, HID), lambda i: (i, 0)),
            pl.BlockSpec((_BLK, 1), lambda i: (i, 0)),
            pl.BlockSpec((_BLK, 1), lambda i: (i, 0)),
        ],
        out_shape=[
            jax.ShapeDtypeStruct((N, HID), jnp.float32),
            jax.ShapeDtypeStruct((N, 1), jnp.float32),
            jax.ShapeDtypeStruct((N, 1), jnp.float32),
        ],
    )(acc, acc, den3, den3, b.reshape(1, HID), w,
      att_s.reshape(HID, 1), att_d.reshape(HID, 1))


def _tc_final_body(acc_ref, den_ref, b_ref, bt_ref, wl_ref, bl_ref, out_ref):
    h = ((acc_ref[0] + acc_ref[1])
         / (den_ref[0] + den_ref[1] + 1e-16) + b_ref[...])
    ids = bt_ref[...]
    gidx = lax.broadcasted_iota(jnp.int32, (G, NP), 0)
    mask = (ids == gidx).astype(jnp.float32)
    sums = jnp.dot(mask, h, preferred_element_type=jnp.float32)
    cnt = jnp.sum(mask, axis=1, keepdims=True)
    pooled = sums / jnp.maximum(cnt, 1.0)
    out_ref[...] = jnp.tanh(
        jnp.dot(pooled, wl_ref[...], preferred_element_type=jnp.float32)
        + bl_ref[...])


def _tc_final(acc, den3, b, batch_p, wl, bl):
    return pl.pallas_call(
        _tc_final_body,
        out_shape=jax.ShapeDtypeStruct((G, 1), jnp.float32),
    )(acc, den3, b.reshape(1, HID), batch_p.reshape(1, NP), wl,
      bl.reshape(1, 1))


def kernel(x, edge_index, edge_attr, batch, W1, att_src1, att_dst1, We1,
           att_e1, b1, W2, att_src2, att_dst2, We2, att_e2, b2, Wl, bl):
    batch_p = jnp.pad(batch, (0, NP - N), constant_values=G)
    ei3 = edge_index.reshape(2, E // K, K)

    h1, as1, ad1, v12 = _tc_node(x, W1, att_src1, att_dst1,
                                 We1.T, att_e1, We2.T, att_e2)
    ae1, ae2 = _tc_edge(edge_attr.reshape(E // K, K, ED), v12)
    acc1, den1 = _sc_edge(ei3, ae1, as1.reshape(N), ad1.reshape(N), h1)

    h2, as2, ad2 = _tc_mid(acc1, den1.reshape(2, NP, 1), b1, W2,
                           att_src2, att_dst2)
    acc2, den2 = _sc_edge(ei3, ae2, as2.reshape(N), ad2.reshape(N), h2)

    return _tc_final(acc2, den2.reshape(2, NP, 1), b2, batch_p, Wl, bl)
